# Initial kernel scaffold; baseline (speedup 1.0000x reference)
import jax
import jax.numpy as jnp
from jax.experimental import pallas as pl


def kernel(x, edge_index, W1, b1, W2, b2, W3, b3, Wl, bl):
    raise NotImplementedError("write your pallas kernel here")


# speedup vs baseline: 2.4793x; 2.4793x over previous
import functools

import jax
import jax.numpy as jnp
from jax import lax
from jax.experimental import pallas as pl
from jax.experimental.pallas import tpu as pltpu
from jax.experimental.pallas import tpu_sc as plsc

_N = 10000
_E = 320000
_D = 128
_NC = 2
_NS = 16
_NW = _NC * _NS
_CHUNK = 128
_NCH = 80
_E_PAD = _NW * _NCH * _CHUNK
_AGG_ROWS = 10240
_RPT = _AGG_ROWS // _NS
_DUMMY = _N
_BLK = 1000


def _sc_agg_body(h_hbm, srcs_hbm, dsts_hbm, zeros_hbm, out_hbm,
                 src_v, dst_v, rows_v, agg_s, sem):
    cid = lax.axis_index("c")
    sid = lax.axis_index("s")
    wid = sid * _NC + cid
    pltpu.sync_copy(zeros_hbm, agg_s.at[pl.ds(sid * _RPT, _RPT)])
    pltpu.sync_copy(srcs_hbm.at[wid], src_v)
    pltpu.sync_copy(dsts_hbm.at[wid], dst_v)
    plsc.subcore_barrier()

    def body(j, carry):
        pltpu.async_copy(h_hbm.at[src_v.at[j]], rows_v, sem).wait()
        pltpu.sync_copy(rows_v, agg_s.at[dst_v.at[j]], add=True)
        return carry

    lax.fori_loop(0, _NCH, body, 0)
    plsc.subcore_barrier()
    pltpu.sync_copy(agg_s.at[pl.ds(sid * _RPT, _RPT)],
                    out_hbm.at[cid].at[pl.ds(sid * _RPT, _RPT)])


_sc_agg = functools.partial(
    pl.kernel,
    out_type=jax.ShapeDtypeStruct((_NC, _AGG_ROWS, _D), jnp.float32),
    mesh=plsc.VectorSubcoreMesh(core_axis_name="c", subcore_axis_name="s",
                                num_cores=_NC, num_subcores=_NS),
    scratch_types=[
        pltpu.VMEM((_NCH, _CHUNK), jnp.int32),
        pltpu.VMEM((_NCH, _CHUNK), jnp.int32),
        pltpu.VMEM((_CHUNK, _D), jnp.float32),
        pltpu.VMEM_SHARED((_AGG_ROWS, _D), jnp.float32),
        pltpu.SemaphoreType.DMA,
    ],
)(_sc_agg_body)


def _fused_kernel(h_ref, p0_ref, p1_ref, b_ref, w_ref, o_ref):
    z = h_ref[...] + (p0_ref[0] + p1_ref[0])
    o_ref[...] = jnp.maximum(
        jnp.dot(z, w_ref[...], preferred_element_type=jnp.float32)
        + b_ref[...], 0.0)


def _fused(h, p, wt, b):
    return pl.pallas_call(
        _fused_kernel,
        grid=(_N // _BLK,),
        in_specs=[
            pl.BlockSpec((_BLK, _D), lambda i: (i, 0)),
            pl.BlockSpec((1, _BLK, _D), lambda i: (0, i, 0)),
            pl.BlockSpec((1, _BLK, _D), lambda i: (1, i, 0)),
            pl.BlockSpec((1, _D), lambda i: (0, 0)),
            pl.BlockSpec((_D, _D), lambda i: (0, 0)),
        ],
        out_specs=pl.BlockSpec((_BLK, _D), lambda i: (i, 0)),
        out_shape=jax.ShapeDtypeStruct((_N, _D), jnp.float32),
    )(h, p, p, b.reshape(1, _D), wt)


def _head_kernel(h_ref, w_ref, b_ref, o_ref):
    o_ref[...] = jnp.dot(h_ref[...], w_ref[...],
                         preferred_element_type=jnp.float32) + b_ref[...]


def _head(h, wt, ob):
    return pl.pallas_call(
        _head_kernel,
        grid=(_N // _BLK,),
        in_specs=[
            pl.BlockSpec((_BLK, _D), lambda i: (i, 0)),
            pl.BlockSpec((_D, _D), lambda i: (0, 0)),
            pl.BlockSpec((1, _D), lambda i: (0, 0)),
        ],
        out_specs=pl.BlockSpec((_BLK, _D), lambda i: (i, 0)),
        out_shape=jax.ShapeDtypeStruct((_N, _D), jnp.float32),
    )(h, wt, ob.reshape(1, _D))


def kernel(x, edge_index, W1, b1, W2, b2, W3, b3, Wl, bl):
    src = edge_index[0].astype(jnp.int32)
    dst = edge_index[1].astype(jnp.int32)
    pad = _E_PAD - _E
    srcs = jnp.concatenate([src, jnp.zeros((pad,), jnp.int32)])
    dsts = jnp.concatenate([dst, jnp.full((pad,), _DUMMY, jnp.int32)])
    srcs = srcs.reshape(_NW, _NCH, _CHUNK)
    dsts = dsts.reshape(_NW, _NCH, _CHUNK)
    zeros = jnp.zeros((_RPT, _D), jnp.float32)
    w1t, w2t, w3t = W1.T, W2.T, W3.T
    wlt = jnp.zeros((_D, _D), jnp.float32).at[:, 0].set(Wl[0])
    obl = jnp.zeros((_D,), jnp.float32).at[0].set(bl[0])

    p = _sc_agg(x, srcs, dsts, zeros)
    h = _fused(x, p, w1t, b1)
    p = _sc_agg(h, srcs, dsts, zeros)
    h = _fused(h, p, w2t, b2)
    p = _sc_agg(h, srcs, dsts, zeros)
    h = _fused(h, p, w3t, b3)
    out = _head(h, wlt, obl)
    return out[:, :1]


# speedup vs baseline: 3.2899x; 1.3269x over previous
import functools

import jax
import jax.numpy as jnp
from jax import lax
from jax.experimental import pallas as pl
from jax.experimental.pallas import tpu as pltpu
from jax.experimental.pallas import tpu_sc as plsc

_N = 10000
_E = 320000
_D = 128
_NC = 2
_NS = 16
_NW = _NC * _NS
_CHUNK = 64
_NCH = 160
_NPASS = 2
_NCHP = _NCH // _NPASS
_E_PAD = _NW * _NCH * _CHUNK
_AGG_ROWS = 10240
_RPT = _AGG_ROWS // _NS
_DUMMY = _N
_BLK = 1000


def _sc_agg_body(h_hbm, srcs_hbm, dsts_hbm, zeros_hbm, out_hbm,
                 src_v, dst_v, rows_a, rows_b, agg_s, sem_a, sem_b):
    cid = lax.axis_index("c")
    sid = lax.axis_index("s")
    wid = sid * _NC + cid
    pltpu.sync_copy(zeros_hbm, agg_s.at[pl.ds(sid * _RPT, _RPT)])
    plsc.subcore_barrier()

    def gather(j, buf, sem):
        return pltpu.make_async_copy(h_hbm.at[src_v.at[j]], buf, sem)

    def scatter(j, buf):
        pltpu.sync_copy(buf, agg_s.at[dst_v.at[j]], add=True)

    for p in range(_NPASS):
        pltpu.sync_copy(srcs_hbm.at[wid].at[p], src_v)
        pltpu.sync_copy(dsts_hbm.at[wid].at[p], dst_v)
        gather(0, rows_a, sem_a).start()

        def body2(k, carry):
            j = 2 * k
            gather(j + 1, rows_b, sem_b).start()
            gather(j, rows_a, sem_a).wait()
            scatter(j, rows_a)
            gather(j + 2, rows_a, sem_a).start()
            gather(j + 1, rows_b, sem_b).wait()
            scatter(j + 1, rows_b)
            return carry

        lax.fori_loop(0, _NCHP // 2 - 1, body2, 0)
        j = _NCHP - 2
        gather(j + 1, rows_b, sem_b).start()
        gather(j, rows_a, sem_a).wait()
        scatter(j, rows_a)
        gather(j + 1, rows_b, sem_b).wait()
        scatter(j + 1, rows_b)
    plsc.subcore_barrier()
    pltpu.sync_copy(agg_s.at[pl.ds(sid * _RPT, _RPT)],
                    out_hbm.at[cid].at[pl.ds(sid * _RPT, _RPT)])


_sc_agg = functools.partial(
    pl.kernel,
    out_type=jax.ShapeDtypeStruct((_NC, _AGG_ROWS, _D), jnp.float32),
    mesh=plsc.VectorSubcoreMesh(core_axis_name="c", subcore_axis_name="s",
                                num_cores=_NC, num_subcores=_NS),
    scratch_types=[
        pltpu.VMEM((_NCHP, _CHUNK), jnp.int32),
        pltpu.VMEM((_NCHP, _CHUNK), jnp.int32),
        pltpu.VMEM((_CHUNK, _D), jnp.float32),
        pltpu.VMEM((_CHUNK, _D), jnp.float32),
        pltpu.VMEM_SHARED((_AGG_ROWS, _D), jnp.float32),
        pltpu.SemaphoreType.DMA,
        pltpu.SemaphoreType.DMA,
    ],
)(_sc_agg_body)


def _fused_kernel(h_ref, p0_ref, p1_ref, b_ref, w_ref, o_ref):
    z = h_ref[...] + (p0_ref[0] + p1_ref[0])
    o_ref[...] = jnp.maximum(
        jnp.dot(z, w_ref[...], preferred_element_type=jnp.float32)
        + b_ref[...], 0.0)


def _fused(h, p, wt, b):
    return pl.pallas_call(
        _fused_kernel,
        grid=(_N // _BLK,),
        in_specs=[
            pl.BlockSpec((_BLK, _D), lambda i: (i, 0)),
            pl.BlockSpec((1, _BLK, _D), lambda i: (0, i, 0)),
            pl.BlockSpec((1, _BLK, _D), lambda i: (1, i, 0)),
            pl.BlockSpec((1, _D), lambda i: (0, 0)),
            pl.BlockSpec((_D, _D), lambda i: (0, 0)),
        ],
        out_specs=pl.BlockSpec((_BLK, _D), lambda i: (i, 0)),
        out_shape=jax.ShapeDtypeStruct((_N, _D), jnp.float32),
    )(h, p, p, b.reshape(1, _D), wt)


def _head_kernel(h_ref, w_ref, b_ref, o_ref):
    o_ref[...] = jnp.dot(h_ref[...], w_ref[...],
                         preferred_element_type=jnp.float32) + b_ref[...]


def _head(h, wt, ob):
    return pl.pallas_call(
        _head_kernel,
        grid=(_N // _BLK,),
        in_specs=[
            pl.BlockSpec((_BLK, _D), lambda i: (i, 0)),
            pl.BlockSpec((_D, _D), lambda i: (0, 0)),
            pl.BlockSpec((1, _D), lambda i: (0, 0)),
        ],
        out_specs=pl.BlockSpec((_BLK, _D), lambda i: (i, 0)),
        out_shape=jax.ShapeDtypeStruct((_N, _D), jnp.float32),
    )(h, wt, ob.reshape(1, _D))


def kernel(x, edge_index, W1, b1, W2, b2, W3, b3, Wl, bl):
    src = edge_index[0].astype(jnp.int32)
    dst = edge_index[1].astype(jnp.int32)
    pad = _E_PAD - _E
    srcs = jnp.concatenate([src, jnp.zeros((pad,), jnp.int32)])
    dsts = jnp.concatenate([dst, jnp.full((pad,), _DUMMY, jnp.int32)])
    srcs = srcs.reshape(_NW, _NPASS, _NCHP, _CHUNK)
    dsts = dsts.reshape(_NW, _NPASS, _NCHP, _CHUNK)
    zeros = jnp.zeros((_RPT, _D), jnp.float32)
    w1t, w2t, w3t = W1.T, W2.T, W3.T
    wlt = jnp.zeros((_D, _D), jnp.float32).at[:, 0].set(Wl[0])
    obl = jnp.zeros((_D,), jnp.float32).at[0].set(bl[0])

    p = _sc_agg(x, srcs, dsts, zeros)
    h = _fused(x, p, w1t, b1)
    p = _sc_agg(h, srcs, dsts, zeros)
    h = _fused(h, p, w2t, b2)
    p = _sc_agg(h, srcs, dsts, zeros)
    h = _fused(h, p, w3t, b3)
    out = _head(h, wlt, obl)
    return out[:, :1]


# speedup vs baseline: 3.6615x; 1.1129x over previous
import functools

import jax
import jax.numpy as jnp
from jax import lax
from jax.experimental import pallas as pl
from jax.experimental.pallas import tpu as pltpu
from jax.experimental.pallas import tpu_sc as plsc

_N = 10000
_E = 320000
_D = 128
_NC = 2
_NS = 16
_NW = _NC * _NS
_CHUNK = 64
_CP = 40
_C0 = 240
_C1 = 80
_NP0 = _C0 // _CP
_NP1 = _C1 // _CP
_NCH = _C0 + _C1
_E_PAD = _NS * _NCH * _CHUNK
_AGG_ROWS = 10240
_RPT = _AGG_ROWS // _NS
_DUMMY = _N
_BLK = 1000


def _sc_agg_body(h_hbm, srcs0_hbm, dsts0_hbm, srcs1_hbm, dsts1_hbm,
                 zeros_hbm, out_hbm,
                 src_v, dst_v, rows_a, rows_b, agg_s, sem_a, sem_b):
    cid = lax.axis_index("c")
    sid = lax.axis_index("s")
    pltpu.sync_copy(zeros_hbm, agg_s.at[pl.ds(sid * _RPT, _RPT)])
    plsc.subcore_barrier()

    def gather(j, buf, sem):
        return pltpu.make_async_copy(h_hbm.at[src_v.at[j]], buf, sem)

    def scatter(j, buf):
        pltpu.sync_copy(buf, agg_s.at[dst_v.at[j]], add=True)

    def run(srcs_hbm, dsts_hbm, npass):
        for p in range(npass):
            pltpu.sync_copy(srcs_hbm.at[sid].at[p], src_v)
            pltpu.sync_copy(dsts_hbm.at[sid].at[p], dst_v)
            gather(0, rows_a, sem_a).start()

            def body2(k, carry):
                j = 2 * k
                gather(j + 1, rows_b, sem_b).start()
                gather(j, rows_a, sem_a).wait()
                scatter(j, rows_a)
                gather(j + 2, rows_a, sem_a).start()
                gather(j + 1, rows_b, sem_b).wait()
                scatter(j + 1, rows_b)
                return carry

            lax.fori_loop(0, _CP // 2 - 1, body2, 0)
            j = _CP - 2
            gather(j + 1, rows_b, sem_b).start()
            gather(j, rows_a, sem_a).wait()
            scatter(j, rows_a)
            gather(j + 1, rows_b, sem_b).wait()
            scatter(j + 1, rows_b)

    @pl.when(cid == 0)
    def _():
        run(srcs0_hbm, dsts0_hbm, _NP0)

    @pl.when(cid == 1)
    def _():
        run(srcs1_hbm, dsts1_hbm, _NP1)
    plsc.subcore_barrier()
    pltpu.sync_copy(agg_s.at[pl.ds(sid * _RPT, _RPT)],
                    out_hbm.at[cid].at[pl.ds(sid * _RPT, _RPT)])


_sc_agg = functools.partial(
    pl.kernel,
    out_type=jax.ShapeDtypeStruct((_NC, _AGG_ROWS, _D), jnp.float32),
    mesh=plsc.VectorSubcoreMesh(core_axis_name="c", subcore_axis_name="s",
                                num_cores=_NC, num_subcores=_NS),
    scratch_types=[
        pltpu.VMEM((_CP, _CHUNK), jnp.int32),
        pltpu.VMEM((_CP, _CHUNK), jnp.int32),
        pltpu.VMEM((_CHUNK, _D), jnp.float32),
        pltpu.VMEM((_CHUNK, _D), jnp.float32),
        pltpu.VMEM_SHARED((_AGG_ROWS, _D), jnp.float32),
        pltpu.SemaphoreType.DMA,
        pltpu.SemaphoreType.DMA,
    ],
)(_sc_agg_body)


def _fused_kernel(h_ref, p0_ref, p1_ref, b_ref, w_ref, o_ref):
    z = h_ref[...] + (p0_ref[0] + p1_ref[0])
    o_ref[...] = jnp.maximum(
        jnp.dot(z, w_ref[...], preferred_element_type=jnp.float32)
        + b_ref[...], 0.0)


def _fused(h, p, wt, b):
    return pl.pallas_call(
        _fused_kernel,
        grid=(_N // _BLK,),
        in_specs=[
            pl.BlockSpec((_BLK, _D), lambda i: (i, 0)),
            pl.BlockSpec((1, _BLK, _D), lambda i: (0, i, 0)),
            pl.BlockSpec((1, _BLK, _D), lambda i: (1, i, 0)),
            pl.BlockSpec((1, _D), lambda i: (0, 0)),
            pl.BlockSpec((_D, _D), lambda i: (0, 0)),
        ],
        out_specs=pl.BlockSpec((_BLK, _D), lambda i: (i, 0)),
        out_shape=jax.ShapeDtypeStruct((_N, _D), jnp.float32),
    )(h, p, p, b.reshape(1, _D), wt)


def _head_kernel(h_ref, w_ref, b_ref, o_ref):
    o_ref[...] = jnp.dot(h_ref[...], w_ref[...],
                         preferred_element_type=jnp.float32) + b_ref[...]


def _head(h, wt, ob):
    return pl.pallas_call(
        _head_kernel,
        grid=(_N // _BLK,),
        in_specs=[
            pl.BlockSpec((_BLK, _D), lambda i: (i, 0)),
            pl.BlockSpec((_D, _D), lambda i: (0, 0)),
            pl.BlockSpec((1, _D), lambda i: (0, 0)),
        ],
        out_specs=pl.BlockSpec((_BLK, _D), lambda i: (i, 0)),
        out_shape=jax.ShapeDtypeStruct((_N, _D), jnp.float32),
    )(h, wt, ob.reshape(1, _D))


def kernel(x, edge_index, W1, b1, W2, b2, W3, b3, Wl, bl):
    src = edge_index[0].astype(jnp.int32)
    dst = edge_index[1].astype(jnp.int32)
    pad = _E_PAD - _E
    srcs = jnp.concatenate([src, jnp.zeros((pad,), jnp.int32)])
    dsts = jnp.concatenate([dst, jnp.full((pad,), _DUMMY, jnp.int32)])
    srcs = srcs.reshape(_NS, _NCH, _CHUNK)
    dsts = dsts.reshape(_NS, _NCH, _CHUNK)
    srcs0 = srcs[:, :_C0].reshape(_NS, _NP0, _CP, _CHUNK)
    dsts0 = dsts[:, :_C0].reshape(_NS, _NP0, _CP, _CHUNK)
    srcs1 = srcs[:, _C0:].reshape(_NS, _NP1, _CP, _CHUNK)
    dsts1 = dsts[:, _C0:].reshape(_NS, _NP1, _CP, _CHUNK)
    zeros = jnp.zeros((_RPT, _D), jnp.float32)
    w1t, w2t, w3t = W1.T, W2.T, W3.T
    wlt = jnp.zeros((_D, _D), jnp.float32).at[:, 0].set(Wl[0])
    obl = jnp.zeros((_D,), jnp.float32).at[0].set(bl[0])

    p = _sc_agg(x, srcs0, dsts0, srcs1, dsts1, zeros)
    h = _fused(x, p, w1t, b1)
    p = _sc_agg(h, srcs0, dsts0, srcs1, dsts1, zeros)
    h = _fused(h, p, w2t, b2)
    p = _sc_agg(h, srcs0, dsts0, srcs1, dsts1, zeros)
    h = _fused(h, p, w3t, b3)
    out = _head(h, wlt, obl)
    return out[:, :1]


# speedup vs baseline: 3.6858x; 1.0066x over previous
import functools

import jax
import jax.numpy as jnp
from jax import lax
from jax.experimental import pallas as pl
from jax.experimental.pallas import tpu as pltpu
from jax.experimental.pallas import tpu_sc as plsc

_N = 10000
_E = 320000
_D = 128
_NC = 2
_NS = 16
_NW = _NC * _NS
_CHUNK = 64
_CP = 40
_C0 = 240
_C1 = 80
_NP0 = _C0 // _CP
_NP1 = _C1 // _CP
_NCH = _C0 + _C1
_E_PAD = _NS * _NCH * _CHUNK
_AGG_ROWS = 10240
_RPT = _AGG_ROWS // _NS
_DUMMY = _N
_BLK = 1000


def _sc_agg_body(h_hbm, srcs0_hbm, dsts0_hbm, srcs1_hbm, dsts1_hbm,
                 zeros_hbm, out_hbm,
                 src_v, dst_v, rows_a, rows_b, agg_s, sem_a, sem_b):
    cid = lax.axis_index("c")
    sid = lax.axis_index("s")
    pltpu.sync_copy(zeros_hbm, agg_s.at[pl.ds(sid * _RPT, _RPT)])
    plsc.subcore_barrier()

    def gather(j, buf, sem):
        return pltpu.make_async_copy(h_hbm.at[src_v.at[j]], buf, sem)

    def scatter(j, buf):
        pltpu.sync_copy(buf, agg_s.at[dst_v.at[j]], add=True)

    def run(srcs_hbm, dsts_hbm, npass):
        for p in range(npass):
            pltpu.sync_copy(srcs_hbm.at[sid].at[p], src_v)
            pltpu.sync_copy(dsts_hbm.at[sid].at[p], dst_v)
            gather(0, rows_a, sem_a).start()

            def body2(k, carry):
                j = 2 * k
                gather(j + 1, rows_b, sem_b).start()
                gather(j, rows_a, sem_a).wait()
                scatter(j, rows_a)
                gather(j + 2, rows_a, sem_a).start()
                gather(j + 1, rows_b, sem_b).wait()
                scatter(j + 1, rows_b)
                return carry

            lax.fori_loop(0, _CP // 2 - 1, body2, 0)
            j = _CP - 2
            gather(j + 1, rows_b, sem_b).start()
            gather(j, rows_a, sem_a).wait()
            scatter(j, rows_a)
            gather(j + 1, rows_b, sem_b).wait()
            scatter(j + 1, rows_b)

    @pl.when(cid == 0)
    def _():
        run(srcs0_hbm, dsts0_hbm, _NP0)

    @pl.when(cid == 1)
    def _():
        run(srcs1_hbm, dsts1_hbm, _NP1)
    plsc.subcore_barrier()
    pltpu.sync_copy(agg_s.at[pl.ds(sid * _RPT, _RPT)],
                    out_hbm.at[cid].at[pl.ds(sid * _RPT, _RPT)])


_sc_agg = functools.partial(
    pl.kernel,
    out_type=jax.ShapeDtypeStruct((_NC, _AGG_ROWS, _D), jnp.float32),
    mesh=plsc.VectorSubcoreMesh(core_axis_name="c", subcore_axis_name="s",
                                num_cores=_NC, num_subcores=_NS),
    scratch_types=[
        pltpu.VMEM((_CP, _CHUNK), jnp.int32),
        pltpu.VMEM((_CP, _CHUNK), jnp.int32),
        pltpu.VMEM((_CHUNK, _D), jnp.float32),
        pltpu.VMEM((_CHUNK, _D), jnp.float32),
        pltpu.VMEM_SHARED((_AGG_ROWS, _D), jnp.float32),
        pltpu.SemaphoreType.DMA,
        pltpu.SemaphoreType.DMA,
    ],
)(_sc_agg_body)


def _fused_kernel(h_ref, p0_ref, p1_ref, b_ref, w_ref, o_ref):
    z = h_ref[...] + (p0_ref[0] + p1_ref[0])
    o_ref[...] = jnp.maximum(
        jnp.dot(z, w_ref[...], preferred_element_type=jnp.float32)
        + b_ref[...], 0.0)


def _fused(h, p, wt, b):
    return pl.pallas_call(
        _fused_kernel,
        grid=(_N // _BLK,),
        in_specs=[
            pl.BlockSpec((_BLK, _D), lambda i: (i, 0)),
            pl.BlockSpec((1, _BLK, _D), lambda i: (0, i, 0)),
            pl.BlockSpec((1, _BLK, _D), lambda i: (1, i, 0)),
            pl.BlockSpec((1, _D), lambda i: (0, 0)),
            pl.BlockSpec((_D, _D), lambda i: (0, 0)),
        ],
        out_specs=pl.BlockSpec((_BLK, _D), lambda i: (i, 0)),
        out_shape=jax.ShapeDtypeStruct((_N, _D), jnp.float32),
    )(h, p, p, b.reshape(1, _D), wt)


def _fused_head_kernel(h_ref, p0_ref, p1_ref, b_ref, w_ref, wl_ref, bl_ref,
                       o_ref):
    z = h_ref[...] + (p0_ref[0] + p1_ref[0])
    h3 = jnp.maximum(
        jnp.dot(z, w_ref[...], preferred_element_type=jnp.float32)
        + b_ref[...], 0.0)
    o_ref[...] = jnp.dot(h3, wl_ref[...],
                         preferred_element_type=jnp.float32) + bl_ref[...]


def _fused_head(h, p, wt, b, wlt, obl):
    return pl.pallas_call(
        _fused_head_kernel,
        grid=(_N // _BLK,),
        in_specs=[
            pl.BlockSpec((_BLK, _D), lambda i: (i, 0)),
            pl.BlockSpec((1, _BLK, _D), lambda i: (0, i, 0)),
            pl.BlockSpec((1, _BLK, _D), lambda i: (1, i, 0)),
            pl.BlockSpec((1, _D), lambda i: (0, 0)),
            pl.BlockSpec((_D, _D), lambda i: (0, 0)),
            pl.BlockSpec((_D, _D), lambda i: (0, 0)),
            pl.BlockSpec((1, _D), lambda i: (0, 0)),
        ],
        out_specs=pl.BlockSpec((_BLK, _D), lambda i: (i, 0)),
        out_shape=jax.ShapeDtypeStruct((_N, _D), jnp.float32),
    )(h, p, p, b.reshape(1, _D), wt, wlt, obl.reshape(1, _D))


def kernel(x, edge_index, W1, b1, W2, b2, W3, b3, Wl, bl):
    src = edge_index[0].astype(jnp.int32)
    dst = edge_index[1].astype(jnp.int32)
    pad = _E_PAD - _E
    srcs = jnp.concatenate([src, jnp.zeros((pad,), jnp.int32)])
    dsts = jnp.concatenate([dst, jnp.full((pad,), _DUMMY, jnp.int32)])
    srcs = srcs.reshape(_NS, _NCH, _CHUNK)
    dsts = dsts.reshape(_NS, _NCH, _CHUNK)
    srcs0 = srcs[:, :_C0].reshape(_NS, _NP0, _CP, _CHUNK)
    dsts0 = dsts[:, :_C0].reshape(_NS, _NP0, _CP, _CHUNK)
    srcs1 = srcs[:, _C0:].reshape(_NS, _NP1, _CP, _CHUNK)
    dsts1 = dsts[:, _C0:].reshape(_NS, _NP1, _CP, _CHUNK)
    zeros = jnp.zeros((_RPT, _D), jnp.float32)
    w1t, w2t, w3t = W1.T, W2.T, W3.T
    wlt = jnp.zeros((_D, _D), jnp.float32).at[:, 0].set(Wl[0])
    obl = jnp.zeros((_D,), jnp.float32).at[0].set(bl[0])

    p = _sc_agg(x, srcs0, dsts0, srcs1, dsts1, zeros)
    h = _fused(x, p, w1t, b1)
    p = _sc_agg(h, srcs0, dsts0, srcs1, dsts1, zeros)
    h = _fused(h, p, w2t, b2)
    p = _sc_agg(h, srcs0, dsts0, srcs1, dsts1, zeros)
    out = _fused_head(h, p, w3t, b3, wlt, obl)
    return out[:, :1]
